# baseline (speedup 1.0000x reference)
import jax
import jax.numpy as jnp
from jax.experimental import pallas as pl
from jax.experimental.pallas import tpu as pltpu

_BR = 128
_TILES = _BR // 8
_C = 1024


def _body(x_ref, out_ref):
    i = pl.program_id(0)
    seq = x_ref.shape[1]
    nch = seq // _C
    iota = jax.lax.broadcasted_iota(
        jnp.int32, (8, _C), 1).astype(jnp.float32)
    ones = jnp.ones((_C, 128), jnp.float32)

    def rowsum(a):
        return jax.lax.dot(a, ones, precision=None,
                           preferred_element_type=jnp.float32)

    lens = []
    for t in range(_TILES):
        lr = None
        for c in range(nch):
            xt = x_ref[t * 8:(t + 1) * 8, c * _C:(c + 1) * _C]
            ind = jnp.where(xt != 0.0, 1.0, 0.0)
            lr = rowsum(ind) if lr is None else lr + rowsum(ind)
        lens.append(lr)

    tot = None
    for t in range(_TILES):
        real_len = lens[t]
        thresh = real_len[:, 0:1]
        rs = None
        prev_tail = x_ref[t * 8:(t + 1) * 8, 0:1]
        for c in range(nch):
            xt = x_ref[t * 8:(t + 1) * 8, c * _C:(c + 1) * _C]
            shifted = jnp.concatenate([prev_tail, xt[:, :_C - 1]], axis=1)
            d = jnp.abs(xt - shifted)
            md = jnp.where(iota < (thresh - float(c * _C)), d, 0.0)
            rs = rowsum(md) if rs is None else rs + rowsum(md)
            prev_tail = xt[:, _C - 1:_C]
        per_row = rs / real_len
        row_id = (jax.lax.broadcasted_iota(jnp.int32, (8, 128), 0)
                  + (i * _BR + t * 8)).astype(jnp.float32)
        per_row = jnp.where(row_id >= 1.0, per_row, 0.0)
        tot = per_row if tot is None else tot + per_row
    tot = jnp.sum(tot, axis=0, keepdims=True)
    out_ref[...] = tot[None]


def _consecutive_loss(x):
    bsz, seq = x.shape
    nb = bsz // _BR
    partials = pl.pallas_call(
        _body,
        grid=(nb,),
        in_specs=[pl.BlockSpec((_BR, seq), lambda i: (i, 0))],
        out_specs=pl.BlockSpec((1, 1, 128), lambda i: (i, 0, 0)),
        out_shape=jax.ShapeDtypeStruct((nb, 1, 128), jnp.float32),
        compiler_params=pltpu.CompilerParams(
            dimension_semantics=("parallel",),
        ),
    )(x)
    return jnp.sum(partials[:, 0, 0]) / bsz


def kernel(x):
    return _consecutive_loss(x)


# speedup vs baseline: 3.1044x; 3.1044x over previous
import functools

import jax
import jax.numpy as jnp
from jax.experimental import pallas as pl
from jax.experimental.pallas import tpu as pltpu

_BR = 64
_T = _BR // 8
_G = 4
_NV = 64


def _tile(x_ref, t, c):
    return x_ref[t * 8:(t + 1) * 8, c * 128:(c + 1) * 128]


def _count_group(x_ref, ts):
    cnts = {t: jnp.zeros((8, 128), jnp.float32) for t in ts}
    for c in range(_NV):
        for t in ts:
            xt = _tile(x_ref, t, c)
            cnts[t] = cnts[t] + jnp.where(xt != 0.0, 1.0, 0.0)
    return [jnp.sum(cnts[t], axis=1, keepdims=True) for t in ts]


def _fast_group(x_ref, ts, iota):
    accs = {t: jnp.zeros((8, 128), jnp.float32) for t in ts}
    prev = {t: _tile(x_ref, t, 0) for t in ts}
    for c in range(_NV):
        for t in ts:
            xt = _tile(x_ref, t, c)
            rolled = pltpu.roll(xt, 1, 1)
            shifted = jnp.where(iota == 0.0, prev[t], rolled)
            accs[t] = accs[t] + jnp.abs(xt - shifted)
            prev[t] = rolled
    return [jnp.sum(accs[t], axis=1, keepdims=True) for t in ts]


def _masked_tile(x_ref, t, real_len, iota):
    rows = slice(t * 8, (t + 1) * 8)

    def body(c, carry):
        acc, prev = carry
        xt = x_ref[rows, pl.ds(c * 128, 128)]
        rolled = pltpu.roll(xt, 1, 1)
        shifted = jnp.where(iota == 0.0, prev, rolled)
        d = jnp.abs(xt - shifted)
        thresh = real_len - (c * 128).astype(jnp.float32)
        acc = acc + jnp.where(iota < thresh, d, 0.0)
        return acc, rolled

    init = (jnp.zeros((8, 128), jnp.float32), x_ref[rows, 0:128])
    acc, _ = jax.lax.fori_loop(0, _NV, body, init)
    return jnp.sum(acc, axis=1, keepdims=True)


def _body(x_ref, out_ref, *, seq):
    i = pl.program_id(0)
    iota = jax.lax.broadcasted_iota(
        jnp.int32, (8, 128), 1).astype(jnp.float32)

    groups = [list(range(g, g + _G)) for g in range(0, _T, _G)]
    lens = []
    for ts in groups:
        lens.extend(_count_group(x_ref, ts))

    m = lens[0]
    for ln in lens[1:]:
        m = jnp.minimum(m, ln)
    all_full = jnp.min(m) == float(seq)

    def fast():
        out = []
        for ts in groups:
            out.extend(_fast_group(x_ref, ts, iota))
        return tuple(out)

    def slow():
        return tuple(_masked_tile(x_ref, t, lens[t], iota)
                     for t in range(_T))

    rowsums = jax.lax.cond(all_full, fast, slow)

    tot = None
    for t in range(_T):
        per_row = rowsums[t] / lens[t]
        row_id = (jax.lax.broadcasted_iota(jnp.int32, (8, 1), 0)
                  + (i * _BR + t * 8)).astype(jnp.float32)
        per_row = jnp.where(row_id >= 1.0, per_row, 0.0)
        tot = per_row if tot is None else tot + per_row
    tot = jnp.sum(tot, axis=0, keepdims=True)
    out_ref[...] = jnp.broadcast_to(tot[None], (1, 1, 128))


def _consecutive_loss(x):
    bsz, seq = x.shape
    nb = bsz // _BR
    partials = pl.pallas_call(
        functools.partial(_body, seq=seq),
        grid=(nb,),
        in_specs=[pl.BlockSpec((_BR, seq), lambda i: (i, 0))],
        out_specs=pl.BlockSpec((1, 1, 128), lambda i: (i, 0, 0)),
        out_shape=jax.ShapeDtypeStruct((nb, 1, 128), jnp.float32),
        compiler_params=pltpu.CompilerParams(
            dimension_semantics=("parallel",),
        ),
    )(x)
    return jnp.sum(partials[:, 0, 0]) / bsz


def kernel(x):
    return _consecutive_loss(x)


# speedup vs baseline: 3.8688x; 1.2462x over previous
import functools

import jax
import jax.numpy as jnp
from jax.experimental import pallas as pl
from jax.experimental.pallas import tpu as pltpu

_BR = 128
_T = _BR // 8
_G = 4
_NV = 64


def _tile(x_ref, t, c):
    return x_ref[t * 8:(t + 1) * 8, c * 128:(c + 1) * 128]


def _count_group(x_ref, ts):
    cnts = {t: jnp.zeros((8, 128), jnp.float32) for t in ts}
    for c in range(_NV):
        for t in ts:
            xt = _tile(x_ref, t, c)
            cnts[t] = cnts[t] + jnp.where(xt != 0.0, 1.0, 0.0)
    return [jnp.sum(cnts[t], axis=1, keepdims=True) for t in ts]


def _fast_group(x_ref, ts, iota):
    accs = {t: jnp.zeros((8, 128), jnp.float32) for t in ts}
    prev = {t: _tile(x_ref, t, 0) for t in ts}
    for c in range(_NV):
        for t in ts:
            xt = _tile(x_ref, t, c)
            rolled = pltpu.roll(xt, 1, 1)
            shifted = jnp.where(iota == 0.0, prev[t], rolled)
            accs[t] = accs[t] + jnp.abs(xt - shifted)
            prev[t] = rolled
    return [jnp.sum(accs[t], axis=1, keepdims=True) for t in ts]


def _masked_tile(x_ref, t, real_len, iota):
    rows = slice(t * 8, (t + 1) * 8)

    def body(c, carry):
        acc, prev = carry
        xt = x_ref[rows, pl.ds(c * 128, 128)]
        rolled = pltpu.roll(xt, 1, 1)
        shifted = jnp.where(iota == 0.0, prev, rolled)
        d = jnp.abs(xt - shifted)
        thresh = real_len - (c * 128).astype(jnp.float32)
        acc = acc + jnp.where(iota < thresh, d, 0.0)
        return acc, rolled

    init = (jnp.zeros((8, 128), jnp.float32), x_ref[rows, 0:128])
    acc, _ = jax.lax.fori_loop(0, _NV, body, init)
    return jnp.sum(acc, axis=1, keepdims=True)


def _body(x_ref, out_ref, *, seq):
    i = pl.program_id(0)
    iota = jax.lax.broadcasted_iota(
        jnp.int32, (8, 128), 1).astype(jnp.float32)

    groups = [list(range(g, g + _G)) for g in range(0, _T, _G)]
    lens = []
    for ts in groups:
        lens.extend(_count_group(x_ref, ts))

    m = lens[0]
    for ln in lens[1:]:
        m = jnp.minimum(m, ln)
    all_full = jnp.min(m) == float(seq)

    def fast():
        out = []
        for ts in groups:
            out.extend(_fast_group(x_ref, ts, iota))
        return tuple(out)

    def slow():
        return tuple(_masked_tile(x_ref, t, lens[t], iota)
                     for t in range(_T))

    rowsums = jax.lax.cond(all_full, fast, slow)

    tot = None
    for t in range(_T):
        per_row = rowsums[t] / lens[t]
        row_id = (jax.lax.broadcasted_iota(jnp.int32, (8, 1), 0)
                  + (i * _BR + t * 8)).astype(jnp.float32)
        per_row = jnp.where(row_id >= 1.0, per_row, 0.0)
        tot = per_row if tot is None else tot + per_row
    tot = jnp.sum(tot, axis=0, keepdims=True)
    out_ref[...] = jnp.broadcast_to(tot[None], (1, 1, 128))


def _consecutive_loss(x):
    bsz, seq = x.shape
    nb = bsz // _BR
    partials = pl.pallas_call(
        functools.partial(_body, seq=seq),
        grid=(nb,),
        in_specs=[pl.BlockSpec((_BR, seq), lambda i: (i, 0))],
        out_specs=pl.BlockSpec((1, 1, 128), lambda i: (i, 0, 0)),
        out_shape=jax.ShapeDtypeStruct((nb, 1, 128), jnp.float32),
        compiler_params=pltpu.CompilerParams(
            dimension_semantics=("parallel",),
        ),
    )(x)
    return jnp.sum(partials[:, 0, 0]) / bsz


def kernel(x):
    return _consecutive_loss(x)


# speedup vs baseline: 5.3678x; 1.3875x over previous
import functools

import jax
import jax.numpy as jnp
from jax.experimental import pallas as pl
from jax.experimental.pallas import tpu as pltpu

_BR = 256
_T = _BR // 8
_G = 4
_NV = 64


def _tile(x_ref, t, c):
    return x_ref[t * 8:(t + 1) * 8, c * 128:(c + 1) * 128]


def _combined_group(x_ref, ts, iota):
    cnts = {t: jnp.zeros((8, 128), jnp.float32) for t in ts}
    accs = {t: jnp.zeros((8, 128), jnp.float32) for t in ts}
    prev = {t: _tile(x_ref, t, 0) for t in ts}
    for c in range(_NV):
        for t in ts:
            xt = _tile(x_ref, t, c)
            rolled = pltpu.roll(xt, 1, 1)
            shifted = jnp.where(iota == 0.0, prev[t], rolled)
            accs[t] = accs[t] + jnp.abs(xt - shifted)
            cnts[t] = cnts[t] + jnp.where(xt != 0.0, 1.0, 0.0)
            prev[t] = rolled
    lens = [jnp.sum(cnts[t], axis=1, keepdims=True) for t in ts]
    rsums = [jnp.sum(accs[t], axis=1, keepdims=True) for t in ts]
    return lens, rsums


def _masked_tile(x_ref, t, real_len, iota):
    rows = slice(t * 8, (t + 1) * 8)

    def body(c, carry):
        acc, prev = carry
        xt = x_ref[rows, pl.ds(c * 128, 128)]
        rolled = pltpu.roll(xt, 1, 1)
        shifted = jnp.where(iota == 0.0, prev, rolled)
        d = jnp.abs(xt - shifted)
        thresh = real_len - (c * 128).astype(jnp.float32)
        acc = acc + jnp.where(iota < thresh, d, 0.0)
        return acc, rolled

    init = (jnp.zeros((8, 128), jnp.float32), x_ref[rows, 0:128])
    acc, _ = jax.lax.fori_loop(0, _NV, body, init)
    return jnp.sum(acc, axis=1, keepdims=True)


def _body(x_ref, out_ref, *, seq):
    i = pl.program_id(0)
    iota = jax.lax.broadcasted_iota(
        jnp.int32, (8, 128), 1).astype(jnp.float32)

    lens, rsums = [], []
    for g in range(0, _T, _G):
        ls, rs = _combined_group(x_ref, list(range(g, g + _G)), iota)
        lens.extend(ls)
        rsums.extend(rs)

    m = lens[0]
    for ln in lens[1:]:
        m = jnp.minimum(m, ln)
    all_full = jnp.min(m) == float(seq)

    rowsums = jax.lax.cond(
        all_full,
        lambda: tuple(rsums),
        lambda: tuple(_masked_tile(x_ref, t, lens[t], iota)
                      for t in range(_T)),
    )

    tot = None
    for t in range(_T):
        per_row = rowsums[t] / lens[t]
        row_id = (jax.lax.broadcasted_iota(jnp.int32, (8, 1), 0)
                  + (i * _BR + t * 8)).astype(jnp.float32)
        per_row = jnp.where(row_id >= 1.0, per_row, 0.0)
        tot = per_row if tot is None else tot + per_row
    tot = jnp.sum(tot, axis=0, keepdims=True)
    out_ref[...] = jnp.broadcast_to(tot[None], (1, 1, 128))


def _consecutive_loss(x):
    bsz, seq = x.shape
    nb = bsz // _BR
    partials = pl.pallas_call(
        functools.partial(_body, seq=seq),
        grid=(nb,),
        in_specs=[pl.BlockSpec((_BR, seq), lambda i: (i, 0))],
        out_specs=pl.BlockSpec((1, 1, 128), lambda i: (i, 0, 0)),
        out_shape=jax.ShapeDtypeStruct((nb, 1, 128), jnp.float32),
        compiler_params=pltpu.CompilerParams(
            dimension_semantics=("parallel",),
        ),
    )(x)
    return jnp.sum(partials[:, 0, 0]) / bsz


def kernel(x):
    return _consecutive_loss(x)
